# Initial kernel scaffold; baseline (speedup 1.0000x reference)
import jax
import jax.numpy as jnp
from jax.experimental import pallas as pl


def kernel(x, edge_index, adj_values):
    raise NotImplementedError("write your pallas kernel here")


# speedup vs baseline: 5.2480x; 5.2480x over previous
import functools

import jax
import jax.numpy as jnp
from jax import lax
from jax.experimental import pallas as pl
from jax.experimental.pallas import tpu as pltpu
from jax.experimental.pallas import tpu_sc as plsc

N = 10000
E = 320000
D = 128
C = 1.0
MIN_NORM = 1e-15
BALL_EPS = 4e-3

NC = 2
NS = 16
NW = NC * NS
CHUNK = 128
NCH = E // CHUNK
CH_PER_W = NCH // NW
CH_REM = NCH - CH_PER_W * NW
NP = 10240
ROWS_PER_TILE = NP // NS

TC_BLOCK = 1000


def _artanh(z):
    z = jnp.clip(z, -1.0 + 1e-7, 1.0 - 1e-7)
    return 0.5 * jnp.log((1.0 + z) / (1.0 - z))


def _logmap0_body(x_ref, o_ref):
    v = x_ref[...]
    nrm = jnp.sqrt(jnp.sum(v * v, axis=1, keepdims=True))
    nrm = jnp.maximum(nrm, MIN_NORM)
    scale = _artanh(nrm) / nrm
    o_ref[...] = scale * v


def _epilogue_body(a_ref, b_ref, o_ref):
    s = a_ref[...] + b_ref[...]
    u_nrm = jnp.maximum(
        jnp.sqrt(jnp.sum(s * s, axis=1, keepdims=True)), MIN_NORM)
    y = jnp.tanh(u_nrm) * s / u_nrm
    y_nrm = jnp.maximum(
        jnp.sqrt(jnp.sum(y * y, axis=1, keepdims=True)), MIN_NORM)
    maxnorm = 1.0 - BALL_EPS
    o_ref[...] = jnp.where(y_nrm > maxnorm, y / y_nrm * maxnorm, y)


def _spmm_body(xt_hbm, src_hbm, dst_hbm, adj_hbm, zeros_hbm, out_hbm,
               src_v, dst_v, adj_v, rows_v, acc, sem):
    cid = lax.axis_index("c")
    sid = lax.axis_index("s")
    wid = cid * NS + sid

    pltpu.sync_copy(zeros_hbm.at[pl.ds(sid * ROWS_PER_TILE, ROWS_PER_TILE)],
                    acc.at[pl.ds(sid * ROWS_PER_TILE, ROWS_PER_TILE)])
    plsc.subcore_barrier()

    n_my_chunks = CH_PER_W + jnp.where(wid < CH_REM, 1, 0)

    def chunk_body(i, carry):
        c = wid + i * NW
        base = c * CHUNK
        pltpu.sync_copy(src_hbm.at[pl.ds(base, CHUNK)], src_v)
        pltpu.sync_copy(dst_hbm.at[pl.ds(base, CHUNK)], dst_v)
        pltpu.sync_copy(adj_hbm.at[pl.ds(base, CHUNK)], adj_v)
        pltpu.async_copy(xt_hbm.at[src_v], rows_v, sem).wait()

        def scale_body(g, _):
            off = pl.multiple_of(g * 16, 16)
            s16 = adj_v[pl.ds(off, 16)]
            for l in range(16):
                e = off + l
                s = s16[l]
                for j in range(D // 16):
                    sl = rows_v[e, pl.ds(j * 16, 16)]
                    rows_v[e, pl.ds(j * 16, 16)] = sl * s
            return _
        lax.fori_loop(0, CHUNK // 16, scale_body, 0, unroll=False)

        pltpu.sync_copy(rows_v, acc.at[dst_v], add=True)
        return carry

    lax.fori_loop(0, n_my_chunks, chunk_body, 0, unroll=False)
    plsc.subcore_barrier()

    row0 = cid * NP + sid * ROWS_PER_TILE
    pltpu.sync_copy(acc.at[pl.ds(sid * ROWS_PER_TILE, ROWS_PER_TILE)],
                    out_hbm.at[pl.ds(row0, ROWS_PER_TILE)])


_spmm = functools.partial(
    pl.kernel,
    out_type=jax.ShapeDtypeStruct((NC * NP, D), jnp.float32),
    mesh=plsc.VectorSubcoreMesh(core_axis_name="c", subcore_axis_name="s",
                                num_cores=NC, num_subcores=NS),
    scratch_types=[
        pltpu.VMEM((CHUNK,), jnp.int32),
        pltpu.VMEM((CHUNK,), jnp.int32),
        pltpu.VMEM((CHUNK,), jnp.float32),
        pltpu.VMEM((CHUNK, D), jnp.float32),
        pltpu.VMEM_SHARED((NP, D), jnp.float32),
        pltpu.SemaphoreType.DMA,
    ],
)(_spmm_body)


def kernel(x, edge_index, adj_values):
    x = x.astype(jnp.float32)
    src = edge_index[0].astype(jnp.int32)
    dst = edge_index[1].astype(jnp.int32)
    adj = adj_values.astype(jnp.float32)

    xt = pl.pallas_call(
        _logmap0_body,
        out_shape=jax.ShapeDtypeStruct((N, D), jnp.float32),
        grid=(N // TC_BLOCK,),
        in_specs=[pl.BlockSpec((TC_BLOCK, D), lambda i: (i, 0))],
        out_specs=pl.BlockSpec((TC_BLOCK, D), lambda i: (i, 0)),
    )(x)

    zeros = jnp.zeros((NP, D), jnp.float32)
    partials = _spmm(xt, src, dst, adj, zeros)

    out = pl.pallas_call(
        _epilogue_body,
        out_shape=jax.ShapeDtypeStruct((N, D), jnp.float32),
        grid=(N // TC_BLOCK,),
        in_specs=[pl.BlockSpec((TC_BLOCK, D), lambda i: (i, 0)),
                  pl.BlockSpec((TC_BLOCK, D), lambda i: (i, 0))],
        out_specs=pl.BlockSpec((TC_BLOCK, D), lambda i: (i, 0)),
    )(partials[:N], partials[NP:NP + N])
    return out


# speedup vs baseline: 8.4363x; 1.6075x over previous
import functools

import jax
import jax.numpy as jnp
from jax import lax
from jax.experimental import pallas as pl
from jax.experimental.pallas import tpu as pltpu
from jax.experimental.pallas import tpu_sc as plsc

N = 10000
E = 320000
D = 128
C = 1.0
MIN_NORM = 1e-15
BALL_EPS = 4e-3

NC = 2
NS = 16
NW = NC * NS
CHUNK = 128
NCH = E // CHUNK
CH_PER_W = NCH // NW
CH_REM = NCH - CH_PER_W * NW
NP = 10240
ROWS_PER_TILE = NP // NS

TC_BLOCK = 1000


def _artanh(z):
    z = jnp.clip(z, -1.0 + 1e-7, 1.0 - 1e-7)
    return 0.5 * jnp.log((1.0 + z) / (1.0 - z))


def _logmap0_body(x_ref, o_ref):
    v = x_ref[...]
    nrm = jnp.sqrt(jnp.sum(v * v, axis=1, keepdims=True))
    nrm = jnp.maximum(nrm, MIN_NORM)
    scale = _artanh(nrm) / nrm
    o_ref[...] = scale * v


def _epilogue_body(a_ref, b_ref, o_ref):
    s = a_ref[0] + b_ref[0]
    u_nrm = jnp.maximum(
        jnp.sqrt(jnp.sum(s * s, axis=1, keepdims=True)), MIN_NORM)
    y = jnp.tanh(u_nrm) * s / u_nrm
    y_nrm = jnp.maximum(
        jnp.sqrt(jnp.sum(y * y, axis=1, keepdims=True)), MIN_NORM)
    maxnorm = 1.0 - BALL_EPS
    o_ref[...] = jnp.where(y_nrm > maxnorm, y / y_nrm * maxnorm, y)


def _spmm_body(xt_hbm, pk_hbm, adj_hbm, zeros_hbm, out_hbm,
               pk_a, pk_b, adj_va, adj_vb, rows_a, rows_b, acc, sem_a, sem_b):
    cid = lax.axis_index("c")
    sid = lax.axis_index("s")
    wid = cid * NS + sid

    pltpu.sync_copy(zeros_hbm.at[pl.ds(sid * ROWS_PER_TILE, ROWS_PER_TILE)],
                    acc.at[pl.ds(sid * ROWS_PER_TILE, ROWS_PER_TILE)])
    plsc.subcore_barrier()

    def stage(pk_v, adj_v, i):
        c = jnp.minimum(wid + i * NW, NCH - 1)
        pltpu.sync_copy(pk_hbm.at[c], pk_v)
        pltpu.sync_copy(adj_hbm.at[c], adj_v)

    def gstart(pk_v, rows_v, sem):
        pltpu.async_copy(xt_hbm.at[pk_v.at[0]], rows_v, sem)

    def gwait(pk_v, rows_v, sem):
        pltpu.make_async_copy(xt_hbm.at[pk_v.at[0]], rows_v, sem).wait()

    def scale(adj_v, rows_v):
        def scale_body(g, carry):
            off = pl.multiple_of(g * 16, 16)
            s16 = adj_v[pl.ds(off, 16)]
            for l in range(16):
                e = off + l
                s = s16[l]
                for j in range(D // 16):
                    sl = rows_v[e, pl.ds(j * 16, 16)]
                    rows_v[e, pl.ds(j * 16, 16)] = sl * s
            return carry
        lax.fori_loop(0, CHUNK // 16, scale_body, 0, unroll=False)

    def scatter(pk_v, rows_v):
        pltpu.sync_copy(rows_v, acc.at[pk_v.at[1]], add=True)

    stage(pk_a, adj_va, 0)
    gstart(pk_a, rows_a, sem_a)

    def body(k, carry):
        i0 = 2 * k
        stage(pk_b, adj_vb, i0 + 1)
        gstart(pk_b, rows_b, sem_b)
        gwait(pk_a, rows_a, sem_a)
        scale(adj_va, rows_a)
        scatter(pk_a, rows_a)
        stage(pk_a, adj_va, i0 + 2)
        gstart(pk_a, rows_a, sem_a)
        gwait(pk_b, rows_b, sem_b)
        scale(adj_vb, rows_b)
        scatter(pk_b, rows_b)
        return carry

    lax.fori_loop(0, CH_PER_W // 2, body, 0, unroll=False)

    gwait(pk_a, rows_a, sem_a)
    scale(adj_va, rows_a)

    @pl.when(wid < CH_REM)
    def _():
        scatter(pk_a, rows_a)

    plsc.subcore_barrier()

    pltpu.sync_copy(acc.at[pl.ds(sid * ROWS_PER_TILE, ROWS_PER_TILE)],
                    out_hbm.at[cid, pl.ds(sid * ROWS_PER_TILE, ROWS_PER_TILE)])


_spmm = functools.partial(
    pl.kernel,
    out_type=jax.ShapeDtypeStruct((NC, NP, D), jnp.float32),
    mesh=plsc.VectorSubcoreMesh(core_axis_name="c", subcore_axis_name="s",
                                num_cores=NC, num_subcores=NS),
    scratch_types=[
        pltpu.VMEM((2, CHUNK), jnp.int32),
        pltpu.VMEM((2, CHUNK), jnp.int32),
        pltpu.VMEM((CHUNK,), jnp.float32),
        pltpu.VMEM((CHUNK,), jnp.float32),
        pltpu.VMEM((CHUNK, D), jnp.float32),
        pltpu.VMEM((CHUNK, D), jnp.float32),
        pltpu.VMEM_SHARED((NP, D), jnp.float32),
        pltpu.SemaphoreType.DMA,
        pltpu.SemaphoreType.DMA,
    ],
)(_spmm_body)


def kernel(x, edge_index, adj_values):
    x = x.astype(jnp.float32)
    src = edge_index[0].astype(jnp.int32)
    dst = edge_index[1].astype(jnp.int32)
    adj2d = adj_values.astype(jnp.float32).reshape(NCH, CHUNK)
    packed = jnp.stack([src.reshape(NCH, CHUNK),
                        dst.reshape(NCH, CHUNK)], axis=1)

    xt = pl.pallas_call(
        _logmap0_body,
        out_shape=jax.ShapeDtypeStruct((N, D), jnp.float32),
        grid=(N // TC_BLOCK,),
        in_specs=[pl.BlockSpec((TC_BLOCK, D), lambda i: (i, 0))],
        out_specs=pl.BlockSpec((TC_BLOCK, D), lambda i: (i, 0)),
    )(x)

    zeros = jnp.zeros((NP, D), jnp.float32)
    partials = _spmm(xt, packed, adj2d, zeros)

    out = pl.pallas_call(
        _epilogue_body,
        out_shape=jax.ShapeDtypeStruct((N, D), jnp.float32),
        grid=(N // TC_BLOCK,),
        in_specs=[pl.BlockSpec((1, TC_BLOCK, D), lambda i: (0, i, 0)),
                  pl.BlockSpec((1, TC_BLOCK, D), lambda i: (1, i, 0))],
        out_specs=pl.BlockSpec((TC_BLOCK, D), lambda i: (i, 0)),
    )(partials, partials)
    return out
